# baseline (speedup 1.0000x reference)
import functools

import jax
import jax.numpy as jnp
from jax.experimental import pallas as pl
from jax.experimental.pallas import tpu as pltpu

D_MODEL_ = 4096
HIDDEN_ = 128
N_EXPERTS_ = 64
TOP_K_ = 8
BLOCK_T = 512


def _router_block(x_ref, w1_ref, w2_ref, w_out, idx_out, logits_out):
    h = jnp.dot(x_ref[...], w1_ref[...], preferred_element_type=jnp.float32)
    h = h * jax.nn.sigmoid(h)
    logits = jnp.dot(h, w2_ref[...], preferred_element_type=jnp.float32)
    logits_out[...] = logits

    t = logits.shape[0]
    lane = jax.lax.broadcasted_iota(jnp.int32, (t, N_EXPERTS_), 1)
    neg = jnp.float32(jnp.finfo(jnp.float32).min)
    cur = logits
    vals = []
    inds = []
    for _ in range(TOP_K_):
        m = jnp.max(cur, axis=1, keepdims=True)
        amax = jnp.min(jnp.where(cur == m, lane, N_EXPERTS_), axis=1,
                       keepdims=True)
        vals.append(m)
        inds.append(amax)
        cur = jnp.where(lane == amax, neg, cur)
    v = jnp.concatenate(vals, axis=1)
    e = jnp.exp(v - v[:, :1])
    w_out[...] = e / jnp.sum(e, axis=1, keepdims=True)
    idx_out[...] = jnp.concatenate(inds, axis=1)


@functools.partial(jax.jit, static_argnames=())
def kernel(hidden_states, W1, W2):
    b, s, d = hidden_states.shape
    n = b * s
    x = hidden_states.reshape(n, d)
    grid = (n // BLOCK_T,)
    weights, idx, logits = pl.pallas_call(
        _router_block,
        grid=grid,
        in_specs=[
            pl.BlockSpec((BLOCK_T, d), lambda i: (i, 0)),
            pl.BlockSpec((d, HIDDEN_), lambda i: (0, 0)),
            pl.BlockSpec((HIDDEN_, N_EXPERTS_), lambda i: (0, 0)),
        ],
        out_specs=[
            pl.BlockSpec((BLOCK_T, TOP_K_), lambda i: (i, 0)),
            pl.BlockSpec((BLOCK_T, TOP_K_), lambda i: (i, 0)),
            pl.BlockSpec((BLOCK_T, N_EXPERTS_), lambda i: (i, 0)),
        ],
        out_shape=[
            jax.ShapeDtypeStruct((n, TOP_K_), jnp.float32),
            jax.ShapeDtypeStruct((n, TOP_K_), jnp.int32),
            jax.ShapeDtypeStruct((n, N_EXPERTS_), jnp.float32),
        ],
        compiler_params=pltpu.CompilerParams(
            dimension_semantics=("arbitrary",),
        ),
    )(x, W1, W2)
    return (weights.reshape(b, s, TOP_K_),
            idx.reshape(b, s, TOP_K_),
            logits.reshape(b, s, N_EXPERTS_))


# speedup vs baseline: 1.3963x; 1.3963x over previous
import functools

import jax
import jax.numpy as jnp
from jax.experimental import pallas as pl
from jax.experimental.pallas import tpu as pltpu

D_MODEL_ = 4096
HIDDEN_ = 128
N_EXPERTS_ = 64
TOP_K_ = 8
BLOCK_T = 512


def _router_block(x_ref, w1_ref, w2_ref, w_out, idx_out, logits_out):
    h = jnp.dot(x_ref[...], w1_ref[...], preferred_element_type=jnp.float32)
    h = h * jax.nn.sigmoid(h)
    logits = jnp.dot(h, w2_ref[...], preferred_element_type=jnp.float32)
    logits_out[...] = logits

    t = logits.shape[0]
    lt = logits.T
    row = jax.lax.broadcasted_iota(jnp.int32, (N_EXPERTS_, t), 0)
    neg = jnp.float32(jnp.finfo(jnp.float32).min)
    cur = lt
    vals = []
    inds = []
    for _ in range(TOP_K_):
        m = jnp.max(cur, axis=0, keepdims=True)
        amax = jnp.min(jnp.where(cur == m, row, N_EXPERTS_), axis=0,
                       keepdims=True)
        vals.append(m)
        inds.append(amax)
        cur = jnp.where(row == amax, neg, cur)
    v = jnp.concatenate(vals, axis=0)
    e = jnp.exp(v - v[:1, :])
    w = e / jnp.sum(e, axis=0, keepdims=True)
    w_out[...] = w.T
    idx_out[...] = jnp.concatenate(inds, axis=0).T


@functools.partial(jax.jit, static_argnames=())
def kernel(hidden_states, W1, W2):
    b, s, d = hidden_states.shape
    n = b * s
    x = hidden_states.reshape(n, d)
    grid = (n // BLOCK_T,)
    weights, idx, logits = pl.pallas_call(
        _router_block,
        grid=grid,
        in_specs=[
            pl.BlockSpec((BLOCK_T, d), lambda i: (i, 0)),
            pl.BlockSpec((d, HIDDEN_), lambda i: (0, 0)),
            pl.BlockSpec((HIDDEN_, N_EXPERTS_), lambda i: (0, 0)),
        ],
        out_specs=[
            pl.BlockSpec((BLOCK_T, TOP_K_), lambda i: (i, 0)),
            pl.BlockSpec((BLOCK_T, TOP_K_), lambda i: (i, 0)),
            pl.BlockSpec((BLOCK_T, N_EXPERTS_), lambda i: (i, 0)),
        ],
        out_shape=[
            jax.ShapeDtypeStruct((n, TOP_K_), jnp.float32),
            jax.ShapeDtypeStruct((n, TOP_K_), jnp.int32),
            jax.ShapeDtypeStruct((n, N_EXPERTS_), jnp.float32),
        ],
        compiler_params=pltpu.CompilerParams(
            dimension_semantics=("arbitrary",),
        ),
    )(x, W1, W2)
    return (weights.reshape(b, s, TOP_K_),
            idx.reshape(b, s, TOP_K_),
            logits.reshape(b, s, N_EXPERTS_))


# speedup vs baseline: 1.5021x; 1.0758x over previous
import functools

import jax
import jax.numpy as jnp
from jax.experimental import pallas as pl
from jax.experimental.pallas import tpu as pltpu

D_MODEL_ = 4096
HIDDEN_ = 128
N_EXPERTS_ = 64
TOP_K_ = 8
BLOCK_T = 1024


def _router_block(x_ref, w1_ref, w2_ref, w_out, idx_out, logits_out):
    h = jnp.dot(x_ref[...], w1_ref[...], preferred_element_type=jnp.float32)
    h = h * jax.nn.sigmoid(h)
    logits = jnp.dot(h, w2_ref[...], preferred_element_type=jnp.float32)
    logits_out[...] = logits

    t = logits.shape[0]
    lt = logits.T
    row = jax.lax.broadcasted_iota(jnp.int32, (N_EXPERTS_, t), 0)
    neg = jnp.float32(jnp.finfo(jnp.float32).min)
    cur = lt
    vals = []
    inds = []
    for _ in range(TOP_K_):
        m = jnp.max(cur, axis=0, keepdims=True)
        amax = jnp.min(jnp.where(cur == m, row, N_EXPERTS_), axis=0,
                       keepdims=True)
        vals.append(m)
        inds.append(amax)
        cur = jnp.where(row == amax, neg, cur)
    v = jnp.concatenate(vals, axis=0)
    e = jnp.exp(v - v[:1, :])
    w = e / jnp.sum(e, axis=0, keepdims=True)
    w_out[...] = w.T
    idx_out[...] = jnp.concatenate(inds, axis=0).T


@functools.partial(jax.jit, static_argnames=())
def kernel(hidden_states, W1, W2):
    b, s, d = hidden_states.shape
    n = b * s
    x = hidden_states.reshape(n, d)
    grid = (n // BLOCK_T,)
    weights, idx, logits = pl.pallas_call(
        _router_block,
        grid=grid,
        in_specs=[
            pl.BlockSpec((BLOCK_T, d), lambda i: (i, 0)),
            pl.BlockSpec((d, HIDDEN_), lambda i: (0, 0)),
            pl.BlockSpec((HIDDEN_, N_EXPERTS_), lambda i: (0, 0)),
        ],
        out_specs=[
            pl.BlockSpec((BLOCK_T, TOP_K_), lambda i: (i, 0)),
            pl.BlockSpec((BLOCK_T, TOP_K_), lambda i: (i, 0)),
            pl.BlockSpec((BLOCK_T, N_EXPERTS_), lambda i: (i, 0)),
        ],
        out_shape=[
            jax.ShapeDtypeStruct((n, TOP_K_), jnp.float32),
            jax.ShapeDtypeStruct((n, TOP_K_), jnp.int32),
            jax.ShapeDtypeStruct((n, N_EXPERTS_), jnp.float32),
        ],
        compiler_params=pltpu.CompilerParams(
            dimension_semantics=("arbitrary",),
        ),
    )(x, W1, W2)
    return (weights.reshape(b, s, TOP_K_),
            idx.reshape(b, s, TOP_K_),
            logits.reshape(b, s, N_EXPERTS_))
